# baseline (speedup 1.0000x reference)
import functools

import jax
import jax.numpy as jnp
from jax import lax
from jax.experimental import pallas as pl
from jax.experimental.pallas import tpu as pltpu
from jax.experimental.pallas import tpu_sc as plsc

_K = 81920
_B = 1024
_R_SC = 256
_C = 1024
_NCHUNK = _K // _C
_NC = 2
_NS = 16
_NW = _NC * _NS
_G = _R_SC // _NW
_L = 16
_UNROLL = 2
_NSLOT = 4

_ROW_BLOCKS = [(s, min(8, _G - s)) for s in range(0, _G, 8)]
_NT = 2 * len(_ROW_BLOCKS)

_BM = 384
_BK = 4096


def _sc_dma_chunk(op, c, slot, row0, wf_hbm, bf_hbm, l0w_hbm, x_buf, w_buf,
                  sems):
    h = pltpu.make_async_copy(
        l0w_hbm.at[:, pl.ds(c * _C, _C)], w_buf.at[slot],
        sems.at[slot, _NT])
    (h.start if op == 0 else h.wait)()
    for t in range(_NT):
        src = wf_hbm if t < len(_ROW_BLOCKS) else bf_hbm
        start, size = _ROW_BLOCKS[t % len(_ROW_BLOCKS)]
        h = pltpu.make_async_copy(
            src.at[pl.ds(row0 + start, size), pl.ds(c * _C, _C)],
            x_buf.at[slot, t, pl.ds(0, size)], sems.at[slot, t])
        (h.start if op == 0 else h.wait)()


def _sc_issue_chunk(c, slot, *a):
    _sc_dma_chunk(0, c, slot, *a)


def _sc_wait_chunk(c, slot, *a):
    _sc_dma_chunk(1, c, slot, *a)


def _sc_compute_chunk(slot, x_buf, w_buf, acc_buf):
    for t in range(_NT):
        color = 0 if t < len(_ROW_BLOCKS) else 1
        start, size = _ROW_BLOCKS[t % len(_ROW_BLOCKS)]
        init = []
        for r in range(size):
            for m in range(4):
                init.append(acc_buf[start + r, color * 4 + m, :])

        def body(j, acc, _t=t, _size=size):
            accl = list(acc)
            for u in range(_UNROLL):
                jj = j * _UNROLL + u
                w = [w_buf[slot, m, pl.ds(jj * _L, _L)] for m in range(4)]
                for r in range(_size):
                    x = x_buf[slot, _t, r, pl.ds(jj * _L, _L)]
                    for m in range(4):
                        accl[r * 4 + m] = accl[r * 4 + m] + x * w[m]
            return tuple(accl)

        acc = lax.fori_loop(0, _C // _L // _UNROLL, body, tuple(init))
        for r in range(size):
            for m in range(4):
                acc_buf[start + r, color * 4 + m, :] = acc[r * 4 + m]


def _sc_l0_body(wf_hbm, bf_hbm, l0w_hbm, out_hbm, x_buf, w_buf, acc_buf,
                sems):
    wid = lax.axis_index("s") * _NC + lax.axis_index("c")
    row0 = (_B - _R_SC) + wid * _G

    zero = jnp.zeros((_L,), jnp.float32)
    for row in range(_G):
        for cm in range(8):
            acc_buf[row, cm, :] = zero

    dma_args = (row0, wf_hbm, bf_hbm, l0w_hbm, x_buf, w_buf, sems)
    for c in range(_NSLOT - 1):
        _sc_issue_chunk(c, c, *dma_args)

    def quad(q, carry):
        c0 = _NSLOT * q
        for u in range(_NSLOT):
            _sc_wait_chunk(c0 + u, u, *dma_args)
            _sc_compute_chunk(u, x_buf, w_buf, acc_buf)

            @pl.when(c0 + u + _NSLOT - 1 < _NCHUNK)
            def _(_c=c0 + u + _NSLOT - 1, _s=(u + _NSLOT - 1) % _NSLOT):
                _sc_issue_chunk(_c, _s, *dma_args)

        return carry

    lax.fori_loop(0, _NCHUNK // _NSLOT, quad, 0)

    pltpu.sync_copy(acc_buf, out_hbm.at[pl.ds(wid * _G, _G)])


def _sc_l0(wf, bf, l0w):
    mesh = plsc.VectorSubcoreMesh(core_axis_name="c", subcore_axis_name="s")
    fn = functools.partial(
        pl.kernel,
        mesh=mesh,
        out_type=jax.ShapeDtypeStruct((_R_SC, 8, _L), jnp.float32),
        scratch_types=[
            pltpu.VMEM((_NSLOT, _NT, 8, _C), jnp.float32),
            pltpu.VMEM((_NSLOT, 4, _C), jnp.float32),
            pltpu.VMEM((_G, 8, _L), jnp.float32),
            pltpu.SemaphoreType.DMA((_NSLOT, _NT + 1)),
        ],
    )(_sc_l0_body)
    return fn(wf, bf, l0w)


def _mlp_tail(wb, t, l0b2, l1w, l1b, l2w, l2b):
    bw = jnp.concatenate([wb[:, 4:], wb[:, :4]], axis=1)
    acc = t * wb + (1.0 - t) * bw + l0b2
    l1_x = jnp.clip(acc, 0.0, 1.0)
    h = jax.lax.dot_general(l1_x, l1w, (((1,), (1,)), ((), ())),
                            preferred_element_type=jnp.float32)
    h = jnp.clip(h + l1b, 0.0, 1.0)
    return jnp.sum(h * l2w, axis=1, keepdims=True) + l2b


def _tc_body(wf_ref, bf_ref, w0w_ref, w0b_ref, turn_ref,
             l0b2_ref, l1w_ref, l1b_ref, l2w_ref, l2b_ref,
             out_ref, accw_ref, accb_ref, *, nk):
    k = pl.program_id(1)

    @pl.when(k == 0)
    def _init():
        accw_ref[...] = jnp.zeros_like(accw_ref)
        accb_ref[...] = jnp.zeros_like(accb_ref)

    dims = (((1,), (1,)), ((), ()))
    accw_ref[...] += jax.lax.dot_general(
        wf_ref[...], w0w_ref[...], dims, preferred_element_type=jnp.float32)
    accb_ref[...] += jax.lax.dot_general(
        bf_ref[...], w0b_ref[...], dims, preferred_element_type=jnp.float32)

    @pl.when(k == nk - 1)
    def _tail():
        wb = accw_ref[...] + accb_ref[...]
        out_ref[...] = _mlp_tail(wb, turn_ref[...], l0b2_ref[...],
                                 l1w_ref[...], l1b_ref[...],
                                 l2w_ref[...], l2b_ref[...])


def _sc_tail_body(acc_ref, sel_ref, turn_ref, l0b2_ref, l1w_ref, l1b_ref,
                  l2w_ref, l2b_ref, out_ref):
    wb = jax.lax.dot_general(
        acc_ref[...], sel_ref[...], (((1,), (0,)), ((), ())),
        preferred_element_type=jnp.float32)
    out_ref[...] = _mlp_tail(wb, turn_ref[...], l0b2_ref[...], l1w_ref[...],
                             l1b_ref[...], l2w_ref[...], l2b_ref[...])


@jax.jit
def kernel(white_features, black_features, turn, score, result,
           l0_w, l0_b, l1_w, l1_b, l2_w, l2_b):
    B, K = white_features.shape
    M = l0_w.shape[0]

    sc_acc = _sc_l0(white_features, black_features, l0_w)
    sc_acc128 = sc_acc.reshape(_R_SC, 8 * _L)

    bm, bk = _BM, _BK
    nm, nk = (B - _R_SC) // bm, K // bk

    zeros = jnp.zeros_like(l0_w)
    w0w = jnp.concatenate([l0_w, zeros], axis=0)
    w0b = jnp.concatenate([zeros, l0_w], axis=0)
    l0b2 = jnp.concatenate([l0_b, l0_b]).reshape(1, 2 * M)
    l1b2 = l1_b.reshape(1, -1)
    l2w2 = l2_w.reshape(1, -1)
    l2b2 = l2_b.reshape(1, 1)

    tc_out = pl.pallas_call(
        functools.partial(_tc_body, nk=nk),
        grid=(nm, nk),
        in_specs=[
            pl.BlockSpec((bm, bk), lambda m, k: (m, k)),
            pl.BlockSpec((bm, bk), lambda m, k: (m, k)),
            pl.BlockSpec((2 * M, bk), lambda m, k: (0, k)),
            pl.BlockSpec((2 * M, bk), lambda m, k: (0, k)),
            pl.BlockSpec((bm, 1), lambda m, k: (m, 0)),
            pl.BlockSpec((1, 2 * M), lambda m, k: (0, 0)),
            pl.BlockSpec(l1_w.shape, lambda m, k: (0, 0)),
            pl.BlockSpec((1, 2 * M), lambda m, k: (0, 0)),
            pl.BlockSpec((1, 2 * M), lambda m, k: (0, 0)),
            pl.BlockSpec((1, 1), lambda m, k: (0, 0)),
        ],
        out_specs=pl.BlockSpec((bm, 1), lambda m, k: (m, 0)),
        out_shape=jax.ShapeDtypeStruct((B - _R_SC, 1), jnp.float32),
        scratch_shapes=[
            pltpu.VMEM((bm, 2 * M), jnp.float32),
            pltpu.VMEM((bm, 2 * M), jnp.float32),
        ],
        compiler_params=pltpu.CompilerParams(
            dimension_semantics=("parallel", "arbitrary"),
        ),
    )(white_features, black_features, w0w, w0b, turn,
      l0b2, l1_w, l1b2, l2w2, l2b2)

    sel = jnp.repeat(jnp.eye(8, dtype=jnp.float32), _L, axis=0)
    sc_out = pl.pallas_call(
        _sc_tail_body,
        out_shape=jax.ShapeDtypeStruct((_R_SC, 1), jnp.float32),
    )(sc_acc128, sel, turn[B - _R_SC:], l0b2, l1_w, l1b2, l2w2, l2b2)

    return jnp.concatenate([tc_out, sc_out], axis=0)


# speedup vs baseline: 1.3176x; 1.3176x over previous
import functools

import jax
import jax.numpy as jnp
from jax.experimental import pallas as pl
from jax.experimental.pallas import tpu as pltpu

_BM = 1024
_BK = 2560


def _nnue_body(wf_ref, bf_ref, w0w_ref, w0b_ref, turn_ref,
               l0b2_ref, l1w_ref, l1b_ref, l2w_ref, l2b_ref,
               out_ref, accw_ref, accb_ref, *, nk):
    k = pl.program_id(1)

    @pl.when(k == 0)
    def _init():
        accw_ref[...] = jnp.zeros_like(accw_ref)
        accb_ref[...] = jnp.zeros_like(accb_ref)

    dims = (((1,), (1,)), ((), ()))
    accw_ref[...] += jax.lax.dot_general(
        wf_ref[...], w0w_ref[...], dims, preferred_element_type=jnp.float32)
    accb_ref[...] += jax.lax.dot_general(
        bf_ref[...], w0b_ref[...], dims, preferred_element_type=jnp.float32)

    @pl.when(k == nk - 1)
    def _tail():
        wb = accw_ref[...] + accb_ref[...]
        bw = jnp.concatenate([wb[:, 4:], wb[:, :4]], axis=1)
        t = turn_ref[...]
        acc = t * wb + (1.0 - t) * bw + l0b2_ref[...]
        l1_x = jnp.clip(acc, 0.0, 1.0)
        h = jax.lax.dot_general(l1_x, l1w_ref[...], (((1,), (0,)), ((), ())),
                                preferred_element_type=jnp.float32)
        h = jnp.clip(h + l1b_ref[...], 0.0, 1.0)
        out_ref[...] = jnp.sum(h * l2w_ref[...], axis=1, keepdims=True) \
            + l2b_ref[...]


@jax.jit
def kernel(white_features, black_features, turn, score, result,
           l0_w, l0_b, l1_w, l1_b, l2_w, l2_b):
    B, K = white_features.shape
    M = l0_w.shape[0]

    bm, bk = _BM, _BK
    nm, nk = B // bm, K // bk

    zeros = jnp.zeros_like(l0_w)
    w0w = jnp.concatenate([l0_w, zeros], axis=0)
    w0b = jnp.concatenate([zeros, l0_w], axis=0)
    l0b2 = jnp.concatenate([l0_b, l0_b]).reshape(1, 2 * M)
    l1b2 = l1_b.reshape(1, -1)
    l2w2 = l2_w.reshape(1, -1)
    l2b2 = l2_b.reshape(1, 1)

    out = pl.pallas_call(
        functools.partial(_nnue_body, nk=nk),
        grid=(nm, nk),
        in_specs=[
            pl.BlockSpec((bm, bk), lambda m, k: (m, k)),
            pl.BlockSpec((bm, bk), lambda m, k: (m, k)),
            pl.BlockSpec((2 * M, bk), lambda m, k: (0, k)),
            pl.BlockSpec((2 * M, bk), lambda m, k: (0, k)),
            pl.BlockSpec((bm, 1), lambda m, k: (m, 0)),
            pl.BlockSpec((1, 2 * M), lambda m, k: (0, 0)),
            pl.BlockSpec(l1_w.T.shape, lambda m, k: (0, 0)),
            pl.BlockSpec((1, 2 * M), lambda m, k: (0, 0)),
            pl.BlockSpec((1, 2 * M), lambda m, k: (0, 0)),
            pl.BlockSpec((1, 1), lambda m, k: (0, 0)),
        ],
        out_specs=pl.BlockSpec((bm, 1), lambda m, k: (m, 0)),
        out_shape=jax.ShapeDtypeStruct((B, 1), jnp.float32),
        scratch_shapes=[
            pltpu.VMEM((bm, 2 * M), jnp.float32),
            pltpu.VMEM((bm, 2 * M), jnp.float32),
        ],
        compiler_params=pltpu.CompilerParams(
            dimension_semantics=("parallel", "arbitrary"),
        ),
    )(white_features, black_features, w0w, w0b, turn,
      l0b2, l1_w.T, l1b2, l2w2, l2b2)
    return out


# speedup vs baseline: 1.3466x; 1.0220x over previous
import functools

import jax
import jax.numpy as jnp
from jax.experimental import pallas as pl
from jax.experimental.pallas import tpu as pltpu

_BM = 1024
_BK = 1024


def _nnue_body(wfa_ref, wfb_ref, bfa_ref, bfb_ref,
               w0wa_ref, w0wb_ref, w0ba_ref, w0bb_ref, turn_ref,
               l0b2_ref, l1w_ref, l1b_ref, l2w_ref, l2b_ref,
               out_ref, accw_ref, accb_ref, *, nk):
    k = pl.program_id(1)

    @pl.when(k == 0)
    def _init():
        accw_ref[...] = jnp.zeros_like(accw_ref)
        accb_ref[...] = jnp.zeros_like(accb_ref)

    dims = (((1,), (1,)), ((), ()))
    accw_ref[...] += (
        jax.lax.dot_general(wfa_ref[...], w0wa_ref[...], dims,
                            preferred_element_type=jnp.float32)
        + jax.lax.dot_general(wfb_ref[...], w0wb_ref[...], dims,
                              preferred_element_type=jnp.float32))
    accb_ref[...] += (
        jax.lax.dot_general(bfa_ref[...], w0ba_ref[...], dims,
                            preferred_element_type=jnp.float32)
        + jax.lax.dot_general(bfb_ref[...], w0bb_ref[...], dims,
                              preferred_element_type=jnp.float32))

    @pl.when(k == nk - 1)
    def _tail():
        wb = accw_ref[...] + accb_ref[...]
        bw = jnp.concatenate([wb[:, 4:], wb[:, :4]], axis=1)
        t = turn_ref[...]
        acc = t * wb + (1.0 - t) * bw + l0b2_ref[...]
        l1_x = jnp.clip(acc, 0.0, 1.0)
        h = jax.lax.dot_general(l1_x, l1w_ref[...], (((1,), (0,)), ((), ())),
                                preferred_element_type=jnp.float32)
        h = jnp.clip(h + l1b_ref[...], 0.0, 1.0)
        out_ref[...] = jnp.sum(h * l2w_ref[...], axis=1, keepdims=True) \
            + l2b_ref[...]


@jax.jit
def kernel(white_features, black_features, turn, score, result,
           l0_w, l0_b, l1_w, l1_b, l2_w, l2_b):
    B, K = white_features.shape
    M = l0_w.shape[0]

    bm, bk = _BM, _BK
    nm, nk = B // bm, K // (2 * bk)

    zeros = jnp.zeros_like(l0_w)
    w0w = jnp.concatenate([l0_w, zeros], axis=0)
    w0b = jnp.concatenate([zeros, l0_w], axis=0)
    l0b2 = jnp.concatenate([l0_b, l0_b]).reshape(1, 2 * M)
    l1b2 = l1_b.reshape(1, -1)
    l2w2 = l2_w.reshape(1, -1)
    l2b2 = l2_b.reshape(1, 1)

    feat_a = pl.BlockSpec((bm, bk), lambda m, k: (m, k))
    feat_b = pl.BlockSpec((bm, bk), lambda m, k, _nk=nk: (m, k + _nk))
    wt_a = pl.BlockSpec((2 * M, bk), lambda m, k: (0, k))
    wt_b = pl.BlockSpec((2 * M, bk), lambda m, k, _nk=nk: (0, k + _nk))

    out = pl.pallas_call(
        functools.partial(_nnue_body, nk=nk),
        grid=(nm, nk),
        in_specs=[
            feat_a, feat_b, feat_a, feat_b,
            wt_a, wt_b, wt_a, wt_b,
            pl.BlockSpec((bm, 1), lambda m, k: (m, 0)),
            pl.BlockSpec((1, 2 * M), lambda m, k: (0, 0)),
            pl.BlockSpec(l1_w.T.shape, lambda m, k: (0, 0)),
            pl.BlockSpec((1, 2 * M), lambda m, k: (0, 0)),
            pl.BlockSpec((1, 2 * M), lambda m, k: (0, 0)),
            pl.BlockSpec((1, 1), lambda m, k: (0, 0)),
        ],
        out_specs=pl.BlockSpec((bm, 1), lambda m, k: (m, 0)),
        out_shape=jax.ShapeDtypeStruct((B, 1), jnp.float32),
        scratch_shapes=[
            pltpu.VMEM((bm, 2 * M), jnp.float32),
            pltpu.VMEM((bm, 2 * M), jnp.float32),
        ],
        compiler_params=pltpu.CompilerParams(
            dimension_semantics=("parallel", "arbitrary"),
        ),
    )(white_features, white_features, black_features, black_features,
      w0w, w0w, w0b, w0b, turn,
      l0b2, l1_w.T, l1b2, l2w2, l2b2)
    return out
